# Initial kernel scaffold; baseline (speedup 1.0000x reference)
import jax
import jax.numpy as jnp
from jax.experimental import pallas as pl


def kernel(inputs, adj, W, a):
    raise NotImplementedError("write your pallas kernel here")


# speedup vs baseline: 2.1015x; 2.1015x over previous
import functools

import jax
import jax.numpy as jnp
from jax import lax
from jax.experimental import pallas as pl
from jax.experimental.pallas import tpu as pltpu
from jax.experimental.pallas import tpu_sc as plsc

N = 10000
D = 128
ALPHA = 0.2
NC = 2
NS = 16
NW = NC * NS
L = 16
BLK = 64
BPC = 16
NROW = 10112
RPT = NROW // NS
NJ = 10
DUMMY = N


def _matmul_body(x_ref, w_ref, o_ref):
    o_ref[...] = jnp.dot(x_ref[...], w_ref[...],
                         preferred_element_type=jnp.float32)


def _matmul(x, W):
    return pl.pallas_call(
        _matmul_body,
        out_shape=jax.ShapeDtypeStruct((N, D), jnp.float32),
    )(x, W)


def _permute(x, idx):
    dnums = lax.GatherDimensionNumbers(
        offset_dims=(), collapsed_slice_dims=(0,), start_index_map=(0,))
    return lax.gather(x, idx[:, None], dnums, (1,),
                      mode=lax.GatherScatterMode.PROMISE_IN_BOUNDS)


def _sc_edges_body(h_hbm, adj_hbm, a_hbm, acc_out, rs_out,
                   src_c, dst_c, ssrc_c, zidx, hs_v, hd_v, sc_v, eev_all, a_v,
                   acc_sp, sem1, sem2, nchunk):
    cid = lax.axis_index("c")
    sid = lax.axis_index("s")
    wid = cid * NS + sid
    row0 = sid * RPT
    iota16 = jnp.arange(L, dtype=jnp.int32)
    zv = jnp.zeros((L,), jnp.float32)

    def zero_scv(r, _):
        for k in range(D // L):
            sc_v[r, pl.ds(k * L, L)] = zv
        return 0

    for j in range(NJ):
        jb = min(j * BLK, RPT - BLK)
        for q in range(BLK // L):
            zidx[j, pl.ds(q * L, L)] = row0 + jb + q * L + iota16

    lax.fori_loop(0, BLK, zero_scv, 0)
    for j in range(NJ):
        pltpu.sync_copy(sc_v, acc_sp.at[zidx.at[j]])

    pltpu.sync_copy(a_hbm, a_v)
    a_ks = [a_v[pl.ds(k * L, L)] for k in range(D // L)]
    perms = [jnp.arange(L, dtype=jnp.int32) ^ sh for sh in (1, 2, 4, 8)]

    plsc.subcore_barrier()

    def chunk_body(c, _):
        pltpu.sync_copy(adj_hbm.at[0, wid, c], src_c)
        pltpu.sync_copy(adj_hbm.at[1, wid, c], dst_c)
        pltpu.sync_copy(adj_hbm.at[2, wid, c], ssrc_c)

        for b in range(BPC):
            cps = pltpu.async_copy(h_hbm.at[src_c.at[b]], hs_v, sem1)
            cpd = pltpu.async_copy(h_hbm.at[dst_c.at[b]], hd_v, sem2)
            cps.wait()
            cpd.wait()
            blk0 = (c * BPC + b) * BLK

            def edge_body(e, grp):
                acc = jnp.zeros((L,), jnp.float32)
                for k in range(D // L):
                    t = hs_v[e, pl.ds(k * L, L)] + hd_v[e, pl.ds(k * L, L)]
                    lr = jnp.where(t > 0.0, t, ALPHA * t)
                    acc = acc + a_ks[k] * lr
                s = acc
                for p in perms:
                    s = s + _permute(s, p)
                eev = jnp.exp(s)
                for k in range(D // L):
                    sc_v[e, pl.ds(k * L, L)] = eev * hd_v[e, pl.ds(k * L, L)]
                grp = jnp.where(iota16 == (e % L), eev, grp)

                @pl.when(e % L == L - 1)
                def _store():
                    gb = pl.multiple_of(blk0 + (e // L) * L, L)
                    eev_all[pl.ds(gb, L)] = grp
                return grp
            lax.fori_loop(0, BLK, edge_body, zv)

            pltpu.sync_copy(sc_v, acc_sp.at[ssrc_c.at[b]], add=True)
        return 0
    lax.fori_loop(0, nchunk, chunk_body, 0)

    plsc.subcore_barrier()

    for j in range(NJ):
        jb = min(j * BLK, RPT - BLK)
        pltpu.async_copy(acc_sp.at[zidx.at[j]], sc_v, sem1).wait()
        pltpu.sync_copy(sc_v, acc_out.at[cid, pl.ds(row0 + jb, BLK)])

    plsc.subcore_barrier()

    lax.fori_loop(0, BLK, zero_scv, 0)
    for j in range(NJ):
        pltpu.sync_copy(sc_v, acc_sp.at[zidx.at[j]])
    plsc.subcore_barrier()

    def chunk2_body(c, _):
        pltpu.sync_copy(adj_hbm.at[2, wid, c], ssrc_c)
        for b in range(BPC):
            blk0 = (c * BPC + b) * BLK

            def e2_body(e, _):
                gb = pl.multiple_of(blk0 + (e // L) * L, L)
                grp = eev_all[pl.ds(gb, L)]
                sel = jnp.where(iota16 == (e % L), grp, 0.0)
                for p in perms:
                    sel = sel + _permute(sel, p)
                for k in range(D // L):
                    sc_v[e, pl.ds(k * L, L)] = sel
                return 0
            lax.fori_loop(0, BLK, e2_body, 0)

            pltpu.sync_copy(sc_v, acc_sp.at[ssrc_c.at[b]], add=True)
        return 0
    lax.fori_loop(0, nchunk, chunk2_body, 0)

    plsc.subcore_barrier()

    for j in range(NJ):
        jb = min(j * BLK, RPT - BLK)
        pltpu.async_copy(acc_sp.at[zidx.at[j]], sc_v, sem1).wait()
        pltpu.sync_copy(sc_v, rs_out.at[cid, pl.ds(row0 + jb, BLK)])


def _sc_edges(h, adj_p, a_flat, nchunk):
    mesh = plsc.VectorSubcoreMesh(core_axis_name="c", subcore_axis_name="s")
    epw = nchunk * BPC * BLK
    f = functools.partial(
        pl.kernel,
        mesh=mesh,
        out_type=[
            jax.ShapeDtypeStruct((NC, NROW, D), jnp.float32),
            jax.ShapeDtypeStruct((NC, NROW, D), jnp.float32),
        ],
        scratch_types=[
            pltpu.VMEM((BPC, BLK), jnp.int32),
            pltpu.VMEM((BPC, BLK), jnp.int32),
            pltpu.VMEM((BPC, BLK), jnp.int32),
            pltpu.VMEM((NJ, BLK), jnp.int32),
            pltpu.VMEM((BLK, D), jnp.float32),
            pltpu.VMEM((BLK, D), jnp.float32),
            pltpu.VMEM((BLK, D), jnp.float32),
            pltpu.VMEM((epw,), jnp.float32),
            pltpu.VMEM((D,), jnp.float32),
            pltpu.VMEM_SHARED((NROW, D), jnp.float32),
            pltpu.SemaphoreType.DMA,
            pltpu.SemaphoreType.DMA,
        ],
    )(functools.partial(_sc_edges_body, nchunk=nchunk))
    return f(h, adj_p, a_flat)


def _final_body(acc_ref, rs_ref, o_ref):
    num = acc_ref[0, pl.ds(0, N), :] + acc_ref[1, pl.ds(0, N), :]
    den = rs_ref[0, pl.ds(0, N), 0:1] + rs_ref[1, pl.ds(0, N), 0:1]
    o_ref[...] = jnp.maximum(num / den, 0.0)


def _final(acc, rs):
    return pl.pallas_call(
        _final_body,
        out_shape=jax.ShapeDtypeStruct((N, D), jnp.float32),
    )(acc, rs)


def kernel(inputs, adj, W, a):
    E = adj.shape[1]
    epc = NW * BPC * BLK
    nchunk = -(-E // epc)
    e_pad = nchunk * epc
    pad = e_pad - E
    gsrc_p = jnp.concatenate([adj[0], jnp.zeros((pad,), jnp.int32)])
    dst_p = jnp.concatenate([adj[1], jnp.zeros((pad,), jnp.int32)])
    ssrc_p = jnp.concatenate([adj[0], jnp.full((pad,), DUMMY, jnp.int32)])
    adj_p = jnp.stack([gsrc_p, dst_p, ssrc_p]).reshape(3, NW, nchunk, BPC, BLK)

    h = _matmul(inputs, W)
    acc, rs = _sc_edges(h, adj_p, a.reshape(D), nchunk)
    return _final(acc, rs)
